# baseline (speedup 1.0000x reference)
import functools
import jax
import jax.numpy as jnp
from jax import lax
from jax.experimental import pallas as pl
from jax.experimental.pallas import tpu as pltpu
from jax.experimental.pallas import tpu_sc as plsc

_N, _C, _B = 32, 81, 8732
_BP = 8736
_NSEG = _BP // 16
_SCALE_XY = 1.0 / 0.1
_SCALE_WH = 1.0 / 0.2
_SPS = 4


def _tc_kernel(pl0_ref, pl1_ref, pl2_ref, pl3_ref, glab_ref, ploc_ref,
               gloc_ref, dbox_ref, con_ref, cn_ref, k_ref, aux_ref):
    g = pl.program_id(0)
    cls = jax.lax.broadcasted_iota(jnp.int32, (_C, _B), 0)
    dxy = dbox_ref[0, 0:2, :]
    dwh = dbox_ref[0, 2:4, :]
    zpad = jnp.zeros((1, _BP - _B), jnp.float32)

    def smooth_l1(d):
        ad = jnp.abs(d)
        return jnp.where(ad < 1.0, 0.5 * d * d, ad - 0.5)

    for j, pref in enumerate((pl0_ref, pl1_ref, pl2_ref, pl3_ref)):
        n = g * _SPS + j
        x = pref[0]
        m = jnp.max(x, axis=0, keepdims=True)
        s = jnp.sum(jnp.exp(x - m), axis=0, keepdims=True)
        lse = m + jnp.log(s)
        glab = glab_ref[pl.ds(n, 1), :]
        true_logit = jnp.sum(jnp.where(cls == glab, x, 0.0), axis=0,
                             keepdims=True)
        con = lse - true_logit
        maskf = (glab > 0).astype(jnp.float32)

        ploc = ploc_ref[n]
        gloc = gloc_ref[n]
        gxy = _SCALE_XY * (gloc[0:2, :] - dxy) / dwh
        gwh = _SCALE_WH * jnp.log(gloc[2:4, :] / dwh)
        sl1 = (jnp.sum(smooth_l1(ploc[0:2, :] - gxy), axis=0, keepdims=True)
               + jnp.sum(smooth_l1(ploc[2:4, :] - gwh), axis=0,
                         keepdims=True))

        pos_i = jnp.sum((glab > 0).astype(jnp.int32))
        base = jnp.sum(maskf * (sl1 + con))
        k1 = jnp.maximum(jnp.minimum(3 * pos_i, _B), 1)
        con_ref[pl.ds(n, 1), :] = jnp.concatenate([con, zpad], axis=1)
        cn_ref[pl.ds(n, 1), :] = jnp.concatenate([con * (1.0 - maskf), zpad],
                                                 axis=1)
        k_ref[pl.ds(n, 1), :] = jnp.full((1, 16), k1, jnp.int32)
        lane = jax.lax.broadcasted_iota(jnp.int32, (1, 16), 1)
        aux_ref[pl.ds(n, 1), :] = jnp.where(
            lane == 0, base,
            jnp.where(lane == 1, jnp.maximum(pos_i.astype(jnp.float32), 1e-6),
                      jnp.where(lane == 2, (pos_i > 0).astype(jnp.float32),
                                0.0)))


def _permute(x, idx):
    return lax.gather(
        x, idx[:, None],
        dimension_numbers=lax.GatherDimensionNumbers(
            offset_dims=(), collapsed_slice_dims=(0,), start_index_map=(0,)),
        slice_sizes=(1,), mode=lax.GatherScatterMode.PROMISE_IN_BOUNDS)


def _tot_all(x):
    lane = lax.iota(jnp.int32, 16)
    for d in (1, 2, 4, 8):
        x = x + _permute(x, lane ^ d)
    return x


def _excl_prefix(x):
    lane = lax.iota(jnp.int32, 16)
    zero = jnp.zeros_like(x)
    inc = x
    for d in (1, 2, 4, 8):
        shifted = _permute(inc, jnp.maximum(lane - d, 0))
        inc = inc + jnp.where(lane >= d, shifted, zero)
    return inc - x


def _sc_mine_kernel(con_hbm, cn_hbm, k_hbm, out_hbm, con_v, cn_v, k_v,
                    out_v):
    wid = lax.axis_index("s") * 2 + lax.axis_index("c")
    pltpu.sync_copy(con_hbm.at[wid], con_v)
    pltpu.sync_copy(cn_hbm.at[wid], cn_v)
    pltpu.sync_copy(k_hbm.at[wid], k_v)
    k1v = k_v[...]
    zi = jnp.zeros((16,), jnp.int32)
    one = jnp.ones((16,), jnp.int32)

    T = zi
    for bit in range(30, -1, -1):
        t = T | jnp.full((16,), 1 << bit, jnp.int32)

        def seg_body(c, cnt, t=t):
            kv = lax.bitcast_convert_type(cn_v[pl.ds(c * 16, 16)], jnp.int32)
            return cnt + jnp.where(kv >= t, one, zi)

        cntv = lax.fori_loop(0, _NSEG, seg_body, zi)
        T = jnp.where(_tot_all(cntv) >= k1v, t, T)

    def cgt_body(c, cnt):
        kv = lax.bitcast_convert_type(cn_v[pl.ds(c * 16, 16)], jnp.int32)
        return cnt + jnp.where(kv > T, one, zi)

    rv = k1v - _tot_all(lax.fori_loop(0, _NSEG, cgt_body, zi))

    def sel_body(c, carry):
        tc, acc = carry
        kv = lax.bitcast_convert_type(cn_v[pl.ds(c * 16, 16)], jnp.int32)
        cv = con_v[pl.ds(c * 16, 16)]
        tiem = kv == T
        ti = jnp.where(tiem, one, zi)
        pref = _excl_prefix(ti) + tc
        sel = (kv > T) | (tiem & (pref < rv))
        return tc + _tot_all(ti), acc + jnp.where(sel, cv, 0.0)

    _, accv = lax.fori_loop(0, _NSEG, sel_body,
                            (zi, jnp.zeros((16,), jnp.float32)))
    out_v[...] = accv
    pltpu.sync_copy(out_v, out_hbm.at[wid])


_sc_mine = functools.partial(
    pl.kernel,
    out_type=jax.ShapeDtypeStruct((_N, 16), jnp.float32),
    mesh=plsc.VectorSubcoreMesh(core_axis_name="c", subcore_axis_name="s"),
    scratch_types=[
        pltpu.VMEM((_BP,), jnp.float32),
        pltpu.VMEM((_BP,), jnp.float32),
        pltpu.VMEM((16,), jnp.int32),
        pltpu.VMEM((16,), jnp.float32),
    ],
)(_sc_mine_kernel)


def kernel(ploc, plabel, gloc, glabel, dboxes):
    con2, cn2, krow, aux = pl.pallas_call(
        _tc_kernel,
        grid=(_N // _SPS,),
        in_specs=[
            pl.BlockSpec((1, _C, _B), lambda g: (_SPS * g + 0, 0, 0)),
            pl.BlockSpec((1, _C, _B), lambda g: (_SPS * g + 1, 0, 0)),
            pl.BlockSpec((1, _C, _B), lambda g: (_SPS * g + 2, 0, 0)),
            pl.BlockSpec((1, _C, _B), lambda g: (_SPS * g + 3, 0, 0)),
            pl.BlockSpec((_N, _B), lambda g: (0, 0)),
            pl.BlockSpec((_N, 4, _B), lambda g: (0, 0, 0)),
            pl.BlockSpec((_N, 4, _B), lambda g: (0, 0, 0)),
            pl.BlockSpec((1, 4, _B), lambda g: (0, 0, 0)),
        ],
        out_specs=[
            pl.BlockSpec((_N, _BP), lambda g: (0, 0)),
            pl.BlockSpec((_N, _BP), lambda g: (0, 0)),
            pl.BlockSpec((_N, 16), lambda g: (0, 0)),
            pl.BlockSpec((_N, 16), lambda g: (0, 0)),
        ],
        out_shape=[
            jax.ShapeDtypeStruct((_N, _BP), jnp.float32),
            jax.ShapeDtypeStruct((_N, _BP), jnp.float32),
            jax.ShapeDtypeStruct((_N, 16), jnp.int32),
            jax.ShapeDtypeStruct((_N, 16), jnp.float32),
        ],
    )(plabel, plabel, plabel, plabel, glabel, ploc, gloc, dboxes)

    negs = jnp.sum(_sc_mine(con2, cn2, krow), axis=1)
    base, pos_f, num_mask = aux[:, 0], aux[:, 1], aux[:, 2]
    return jnp.mean((base + negs) * num_mask / pos_f)


# speedup vs baseline: 1.6545x; 1.6545x over previous
import jax
import jax.numpy as jnp
from jax.experimental import pallas as pl
from jax.experimental.pallas import tpu as pltpu

_N, _C, _B = 32, 81, 8732
_SCALE_XY = 1.0 / 0.1
_SCALE_WH = 1.0 / 0.2
_SPS = 4


def _mine(con, maskf, sl1m, out_ref):
    pos_i = jnp.sum((maskf > 0.5).astype(jnp.int32), axis=1, keepdims=True)
    sl1_sum = jnp.sum(sl1m, axis=1, keepdims=True)
    posc_sum = jnp.sum(con * maskf, axis=1, keepdims=True)

    conneg = con * (1.0 - maskf)
    key = jax.lax.bitcast_convert_type(conneg, jnp.int32)
    k = jnp.minimum(3 * pos_i, _B)
    k1 = jnp.maximum(k, 1)

    def radix_body(i, t_acc):
        cand = t_acc | (jnp.int32(1) << (30 - i))
        cnt = jnp.sum((key >= cand).astype(jnp.int32), axis=1, keepdims=True)
        return jnp.where(cnt >= k1, cand, t_acc)

    T = jax.lax.fori_loop(0, 31, radix_body, jnp.zeros((_N, 1), jnp.int32))
    c_gt = jnp.sum((key > T).astype(jnp.int32), axis=1, keepdims=True)
    r = k1 - c_gt
    tie = key == T
    idx = jax.lax.broadcasted_iota(jnp.int32, (_N, _B), 1)

    def idx_body(i, i_acc):
        cand = i_acc | (jnp.int32(1) << (13 - i))
        cnt = jnp.sum((tie & (idx < cand)).astype(jnp.int32), axis=1,
                      keepdims=True)
        return jnp.where(cnt < r, cand, i_acc)

    ihi = jax.lax.fori_loop(0, 14, idx_body, jnp.zeros((_N, 1), jnp.int32))
    sel = (key > T) | (tie & (idx < ihi + 1) & (r > 0))
    neg_sum = jnp.sum(jnp.where(sel, con, 0.0), axis=1, keepdims=True)

    total = sl1_sum + posc_sum + neg_sum
    num_mask = (pos_i > 0).astype(jnp.float32)
    pos_f = jnp.maximum(pos_i.astype(jnp.float32), 1e-6)
    out_ref[...] = (jnp.sum(total * num_mask / pos_f) / _N).reshape(1, 1)


def _fused_kernel(pl0_ref, pl1_ref, pl2_ref, pl3_ref, glab_ref, ploc_ref,
                  gloc_ref, dbox_ref, out_ref, con_s, mask_s, sl1_s):
    g = pl.program_id(0)
    cls = jax.lax.broadcasted_iota(jnp.int32, (_C, _B), 0)
    dxy = dbox_ref[0, 0:2, :]
    dwh = dbox_ref[0, 2:4, :]

    def smooth_l1(d):
        ad = jnp.abs(d)
        return jnp.where(ad < 1.0, 0.5 * d * d, ad - 0.5)

    for j, pref in enumerate((pl0_ref, pl1_ref, pl2_ref, pl3_ref)):
        n = g * _SPS + j
        x = pref[0]
        m = jnp.max(x, axis=0, keepdims=True)
        s = jnp.sum(jnp.exp(x - m), axis=0, keepdims=True)
        lse = m + jnp.log(s)
        glab = glab_ref[pl.ds(n, 1), :]
        true_logit = jnp.sum(jnp.where(cls == glab, x, 0.0), axis=0,
                             keepdims=True)
        con = lse - true_logit
        maskf = (glab > 0).astype(jnp.float32)

        ploc = ploc_ref[n]
        gloc = gloc_ref[n]
        gxy = _SCALE_XY * (gloc[0:2, :] - dxy) / dwh
        gwh = _SCALE_WH * jnp.log(gloc[2:4, :] / dwh)
        sl1 = (jnp.sum(smooth_l1(ploc[0:2, :] - gxy), axis=0, keepdims=True)
               + jnp.sum(smooth_l1(ploc[2:4, :] - gwh), axis=0,
                         keepdims=True))
        con_s[pl.ds(n, 1), :] = con
        mask_s[pl.ds(n, 1), :] = maskf
        sl1_s[pl.ds(n, 1), :] = maskf * sl1

    @pl.when(g == _N // _SPS - 1)
    def _():
        _mine(con_s[...], mask_s[...], sl1_s[...], out_ref)


def kernel(ploc, plabel, gloc, glabel, dboxes):
    out = pl.pallas_call(
        _fused_kernel,
        grid=(_N // _SPS,),
        in_specs=[
            pl.BlockSpec((1, _C, _B), lambda g: (_SPS * g + 0, 0, 0)),
            pl.BlockSpec((1, _C, _B), lambda g: (_SPS * g + 1, 0, 0)),
            pl.BlockSpec((1, _C, _B), lambda g: (_SPS * g + 2, 0, 0)),
            pl.BlockSpec((1, _C, _B), lambda g: (_SPS * g + 3, 0, 0)),
            pl.BlockSpec((_N, _B), lambda g: (0, 0)),
            pl.BlockSpec((_N, 4, _B), lambda g: (0, 0, 0)),
            pl.BlockSpec((_N, 4, _B), lambda g: (0, 0, 0)),
            pl.BlockSpec((1, 4, _B), lambda g: (0, 0, 0)),
        ],
        out_specs=pl.BlockSpec((1, 1), lambda g: (0, 0)),
        out_shape=jax.ShapeDtypeStruct((1, 1), jnp.float32),
        scratch_shapes=[
            pltpu.VMEM((_N, _B), jnp.float32),
            pltpu.VMEM((_N, _B), jnp.float32),
            pltpu.VMEM((_N, _B), jnp.float32),
        ],
    )(plabel, plabel, plabel, plabel, glabel, ploc, gloc, dboxes)
    return out[0, 0]
